# baseline (speedup 1.0000x reference)
import functools

import jax
import jax.numpy as jnp
from jax.experimental import pallas as pl
from jax.experimental.pallas import tpu as pltpu

_BU = 400
_BV = 512


def _dot(a, b):
    return jax.lax.dot_general(a, b, (((1,), (0,)), ((), ())),
                               preferred_element_type=jnp.float32)


def _dot_t(a, b):
    return jax.lax.dot_general(a, b, (((0,), (0,)), ((), ())),
                               preferred_element_type=jnp.float32)


def _pass_a(adj_ref, hu0_ref, w0_ref, abf_ref, du_ref, cs_ref, acc_ref):
    bu, n_v = adj_ref.shape
    ch = 512 if n_v % 512 == 0 else n_v
    rs = jnp.zeros((bu, 1), jnp.float32)
    cs_parts = []
    for c in range(0, n_v, ch):
        xc = adj_ref[:, c:c + ch]
        abf_ref[:, c:c + ch] = xc.astype(jnp.bfloat16)
        rs = rs + jnp.sum(xc, axis=1, keepdims=True)
        cs_parts.append(jnp.sum(xc, axis=0, keepdims=True))

    du = jax.lax.rsqrt(rs + 1.0)
    du_ref[...] = du
    t0 = _dot(du * hu0_ref[...], w0_ref[...]).astype(jnp.bfloat16)

    @pl.when(pl.program_id(0) == 0)
    def _():
        acc_ref[...] = jnp.zeros_like(acc_ref)
        cs_ref[...] = jnp.zeros_like(cs_ref)

    cs_ref[...] += jnp.concatenate(cs_parts, axis=1)
    acc_ref[...] += _dot_t(abf_ref[...], t0)


def _pass_a2(acc_ref, cs_ref, hv0_ref, w0_ref, w1_ref, dv_ref, swb_ref):
    dv = jax.lax.rsqrt(cs_ref[...] + 1.0).T
    dv_ref[...] = dv
    hv1 = jnp.maximum(dv * acc_ref[...], 0.0)
    s0 = _dot(dv * hv0_ref[...], w0_ref[...])
    s1 = _dot(dv * hv1, w1_ref[...])
    swb_ref[...] = jnp.concatenate([s0, s1], axis=1).astype(jnp.bfloat16)


def _pass_b1(abf_ref, swb_ref, du_ref, w1_ref, hu2_ref, t1_ref):
    d = w1_ref.shape[1]
    z = _dot(abf_ref[...], swb_ref[...])
    du = du_ref[...]
    hu1 = jnp.maximum(du * z[:, :d], 0.0)
    hu2_ref[...] = jnp.maximum(du * z[:, d:], 0.0)
    t1_ref[...] = _dot(du * hu1, w1_ref[...]).astype(jnp.bfloat16)


def _pass_b2(abf_ref, t1_ref, dv_ref, w2_ref, logit_ref):
    n_u, bv = abf_ref.shape
    d = w2_ref.shape[1]
    ch = 2000 if n_u % 2000 == 0 else n_u

    agg = jnp.zeros((bv, d), jnp.float32)
    for r in range(0, n_u, ch):
        agg = agg + _dot_t(abf_ref[r:r + ch, :], t1_ref[r:r + ch, :])
    hv2 = jnp.maximum(dv_ref[...] * agg, 0.0)
    sw2 = _dot(dv_ref[...] * hv2, w2_ref[...]).astype(jnp.bfloat16)

    @pl.when(pl.program_id(0) == 0)
    def _():
        logit_ref[...] = jnp.zeros_like(logit_ref)

    for r in range(0, n_u, ch):
        logit_ref[r:r + ch, :] += _dot(abf_ref[r:r + ch, :], sw2)


def _pass_d(logit_ref, du_ref, out_ref, *, n_cls):
    logits = du_ref[...] * logit_ref[...]
    col = jax.lax.broadcasted_iota(jnp.int32, logits.shape, 1)
    mask = col < n_cls
    masked = jnp.where(mask, logits, -1e30)
    m = jnp.max(masked, axis=1, keepdims=True)
    e = jnp.where(mask, jnp.exp(logits - m), 0.0)
    s = jnp.sum(e, axis=1, keepdims=True)
    out_ref[...] = (logits - m - jnp.log(s))[:, :n_cls]


def kernel(adj, features_u, features_v, W0, W1, W2):
    n_u, n_v = adj.shape
    d_h = W0.shape[1]
    n_cls = W2.shape[1]
    bu = _BU if n_u % _BU == 0 else n_u
    bv = _BV if n_v % _BV == 0 else n_v
    grid = (n_u // bu,)
    f32 = jnp.float32

    abf, du, cs, acc = pl.pallas_call(
        _pass_a,
        grid=grid,
        in_specs=[
            pl.BlockSpec((bu, n_v), lambda i: (i, 0)),
            pl.BlockSpec((bu, d_h), lambda i: (i, 0)),
            pl.BlockSpec((d_h, d_h), lambda i: (0, 0)),
        ],
        out_specs=[
            pl.BlockSpec((bu, n_v), lambda i: (i, 0)),
            pl.BlockSpec((bu, 1), lambda i: (i, 0)),
            pl.BlockSpec((1, n_v), lambda i: (0, 0)),
            pl.BlockSpec((n_v, d_h), lambda i: (0, 0)),
        ],
        out_shape=[
            jax.ShapeDtypeStruct((n_u, n_v), jnp.bfloat16),
            jax.ShapeDtypeStruct((n_u, 1), f32),
            jax.ShapeDtypeStruct((1, n_v), f32),
            jax.ShapeDtypeStruct((n_v, d_h), f32),
        ],
    )(adj, features_u, W0)

    dv, swb = pl.pallas_call(
        _pass_a2,
        in_specs=[
            pl.BlockSpec((n_v, d_h), lambda: (0, 0)),
            pl.BlockSpec((1, n_v), lambda: (0, 0)),
            pl.BlockSpec((n_v, d_h), lambda: (0, 0)),
            pl.BlockSpec((d_h, d_h), lambda: (0, 0)),
            pl.BlockSpec((d_h, d_h), lambda: (0, 0)),
        ],
        out_specs=[
            pl.BlockSpec((n_v, 1), lambda: (0, 0)),
            pl.BlockSpec((n_v, 2 * d_h), lambda: (0, 0)),
        ],
        out_shape=[
            jax.ShapeDtypeStruct((n_v, 1), f32),
            jax.ShapeDtypeStruct((n_v, 2 * d_h), jnp.bfloat16),
        ],
    )(acc, cs, features_v, W0, W1)

    hu2, t1 = pl.pallas_call(
        _pass_b1,
        grid=grid,
        in_specs=[
            pl.BlockSpec((bu, n_v), lambda i: (i, 0)),
            pl.BlockSpec((n_v, 2 * d_h), lambda i: (0, 0)),
            pl.BlockSpec((bu, 1), lambda i: (i, 0)),
            pl.BlockSpec((d_h, d_h), lambda i: (0, 0)),
        ],
        out_specs=[
            pl.BlockSpec((bu, d_h), lambda i: (i, 0)),
            pl.BlockSpec((bu, d_h), lambda i: (i, 0)),
        ],
        out_shape=[
            jax.ShapeDtypeStruct((n_u, d_h), f32),
            jax.ShapeDtypeStruct((n_u, d_h), jnp.bfloat16),
        ],
    )(abf, swb, du, W1)

    w2p = jnp.zeros((d_h, d_h), f32).at[:, :n_cls].set(W2)

    logit = pl.pallas_call(
        _pass_b2,
        grid=(n_v // bv,),
        in_specs=[
            pl.BlockSpec((n_u, bv), lambda j: (0, j)),
            pl.BlockSpec((n_u, d_h), lambda j: (0, 0)),
            pl.BlockSpec((bv, 1), lambda j: (j, 0)),
            pl.BlockSpec((d_h, d_h), lambda j: (0, 0)),
        ],
        out_specs=pl.BlockSpec((n_u, d_h), lambda j: (0, 0)),
        out_shape=jax.ShapeDtypeStruct((n_u, d_h), f32),
    )(abf, t1, dv, w2p)

    logp = pl.pallas_call(
        functools.partial(_pass_d, n_cls=n_cls),
        grid=grid,
        in_specs=[
            pl.BlockSpec((bu, d_h), lambda i: (i, 0)),
            pl.BlockSpec((bu, 1), lambda i: (i, 0)),
        ],
        out_specs=pl.BlockSpec((bu, n_cls), lambda i: (i, 0)),
        out_shape=jax.ShapeDtypeStruct((n_u, n_cls), f32),
    )(logit, du)

    return logp, hu2


# speedup vs baseline: 1.2082x; 1.2082x over previous
import functools

import jax
import jax.numpy as jnp
from jax.experimental import pallas as pl
from jax.experimental.pallas import tpu as pltpu

_BU = 400
_BV = 256


def _dot(a, b):
    return jax.lax.dot_general(a, b, (((1,), (0,)), ((), ())),
                               preferred_element_type=jnp.float32)


def _dot_t(a, b):
    return jax.lax.dot_general(a, b, (((0,), (0,)), ((), ())),
                               preferred_element_type=jnp.float32)


def _pass_a(adj_ref, hu0_ref, w0_ref, abf_ref, du_ref, cs_ref, acc_ref):
    bu, n_v = adj_ref.shape
    ch = 512 if n_v % 512 == 0 else n_v
    rs = jnp.zeros((bu, 1), jnp.float32)
    cs_parts = []
    for c in range(0, n_v, ch):
        xc = adj_ref[:, c:c + ch]
        abf_ref[:, c:c + ch] = xc.astype(jnp.bfloat16)
        rs = rs + jnp.sum(xc, axis=1, keepdims=True)
        cs_parts.append(jnp.sum(xc, axis=0, keepdims=True))

    du = jax.lax.rsqrt(rs + 1.0)
    du_ref[...] = du
    t0 = _dot(du * hu0_ref[...], w0_ref[...]).astype(jnp.bfloat16)

    @pl.when(pl.program_id(0) == 0)
    def _():
        acc_ref[...] = jnp.zeros_like(acc_ref)
        cs_ref[...] = jnp.zeros_like(cs_ref)

    cs_ref[...] += jnp.concatenate(cs_parts, axis=1)
    acc_ref[...] += _dot_t(abf_ref[...], t0)


def _pass_a2(acc_ref, cs_ref, hv0_ref, w0_ref, w1_ref, dv_ref, swb_ref):
    dv = jax.lax.rsqrt(cs_ref[...] + 1.0).T
    dv_ref[...] = dv
    hv1 = jnp.maximum(dv * acc_ref[...], 0.0)
    s0 = _dot(dv * hv0_ref[...], w0_ref[...])
    s1 = _dot(dv * hv1, w1_ref[...])
    swb_ref[...] = jnp.concatenate([s0, s1], axis=1).astype(jnp.bfloat16)


def _pass_b1(abf_ref, swb_ref, du_ref, w1_ref, hu2_ref, t1_ref):
    d = w1_ref.shape[1]
    z = _dot(abf_ref[...], swb_ref[...])
    du = du_ref[...]
    hu1 = jnp.maximum(du * z[:, :d], 0.0)
    hu2_ref[...] = jnp.maximum(du * z[:, d:], 0.0)
    t1_ref[...] = _dot(du * hu1, w1_ref[...]).astype(jnp.bfloat16)


def _pass_b2(abf_ref, t1_ref, dv_ref, w2_ref, logit_ref):
    n_u, bv = abf_ref.shape
    d = w2_ref.shape[1]
    ch = 2000 if n_u % 2000 == 0 else n_u

    agg = jnp.zeros((bv, d), jnp.float32)
    for r in range(0, n_u, ch):
        agg = agg + _dot_t(abf_ref[r:r + ch, :], t1_ref[r:r + ch, :])
    hv2 = jnp.maximum(dv_ref[...] * agg, 0.0)
    sw2 = _dot(dv_ref[...] * hv2, w2_ref[...]).astype(jnp.bfloat16)

    @pl.when(pl.program_id(0) == 0)
    def _():
        logit_ref[...] = jnp.zeros_like(logit_ref)

    for r in range(0, n_u, ch):
        logit_ref[r:r + ch, :] += _dot(abf_ref[r:r + ch, :], sw2)


def _pass_d(logit_ref, du_ref, out_ref, *, n_cls):
    logits = du_ref[...] * logit_ref[...]
    col = jax.lax.broadcasted_iota(jnp.int32, logits.shape, 1)
    mask = col < n_cls
    masked = jnp.where(mask, logits, -1e30)
    m = jnp.max(masked, axis=1, keepdims=True)
    e = jnp.where(mask, jnp.exp(logits - m), 0.0)
    s = jnp.sum(e, axis=1, keepdims=True)
    out_ref[...] = (logits - m - jnp.log(s))[:, :n_cls]


def kernel(adj, features_u, features_v, W0, W1, W2):
    n_u, n_v = adj.shape
    d_h = W0.shape[1]
    n_cls = W2.shape[1]
    bu = _BU if n_u % _BU == 0 else n_u
    bv = _BV if n_v % _BV == 0 else n_v
    grid = (n_u // bu,)
    f32 = jnp.float32

    abf, du, cs, acc = pl.pallas_call(
        _pass_a,
        grid=grid,
        in_specs=[
            pl.BlockSpec((bu, n_v), lambda i: (i, 0)),
            pl.BlockSpec((bu, d_h), lambda i: (i, 0)),
            pl.BlockSpec((d_h, d_h), lambda i: (0, 0)),
        ],
        out_specs=[
            pl.BlockSpec((bu, n_v), lambda i: (i, 0)),
            pl.BlockSpec((bu, 1), lambda i: (i, 0)),
            pl.BlockSpec((1, n_v), lambda i: (0, 0)),
            pl.BlockSpec((n_v, d_h), lambda i: (0, 0)),
        ],
        out_shape=[
            jax.ShapeDtypeStruct((n_u, n_v), jnp.bfloat16),
            jax.ShapeDtypeStruct((n_u, 1), f32),
            jax.ShapeDtypeStruct((1, n_v), f32),
            jax.ShapeDtypeStruct((n_v, d_h), f32),
        ],
    )(adj, features_u, W0)

    dv, swb = pl.pallas_call(
        _pass_a2,
        in_specs=[
            pl.BlockSpec((n_v, d_h), lambda: (0, 0)),
            pl.BlockSpec((1, n_v), lambda: (0, 0)),
            pl.BlockSpec((n_v, d_h), lambda: (0, 0)),
            pl.BlockSpec((d_h, d_h), lambda: (0, 0)),
            pl.BlockSpec((d_h, d_h), lambda: (0, 0)),
        ],
        out_specs=[
            pl.BlockSpec((n_v, 1), lambda: (0, 0)),
            pl.BlockSpec((n_v, 2 * d_h), lambda: (0, 0)),
        ],
        out_shape=[
            jax.ShapeDtypeStruct((n_v, 1), f32),
            jax.ShapeDtypeStruct((n_v, 2 * d_h), jnp.bfloat16),
        ],
    )(acc, cs, features_v, W0, W1)

    hu2, t1 = pl.pallas_call(
        _pass_b1,
        grid=grid,
        in_specs=[
            pl.BlockSpec((bu, n_v), lambda i: (i, 0)),
            pl.BlockSpec((n_v, 2 * d_h), lambda i: (0, 0)),
            pl.BlockSpec((bu, 1), lambda i: (i, 0)),
            pl.BlockSpec((d_h, d_h), lambda i: (0, 0)),
        ],
        out_specs=[
            pl.BlockSpec((bu, d_h), lambda i: (i, 0)),
            pl.BlockSpec((bu, d_h), lambda i: (i, 0)),
        ],
        out_shape=[
            jax.ShapeDtypeStruct((n_u, d_h), f32),
            jax.ShapeDtypeStruct((n_u, d_h), jnp.bfloat16),
        ],
    )(abf, swb, du, W1)

    w2p = jnp.zeros((d_h, d_h), f32).at[:, :n_cls].set(W2)

    logit = pl.pallas_call(
        _pass_b2,
        grid=(n_v // bv,),
        in_specs=[
            pl.BlockSpec((n_u, bv), lambda j: (0, j)),
            pl.BlockSpec((n_u, d_h), lambda j: (0, 0)),
            pl.BlockSpec((bv, 1), lambda j: (j, 0)),
            pl.BlockSpec((d_h, d_h), lambda j: (0, 0)),
        ],
        out_specs=pl.BlockSpec((n_u, d_h), lambda j: (0, 0)),
        out_shape=jax.ShapeDtypeStruct((n_u, d_h), f32),
    )(abf, t1, dv, w2p)

    logp = pl.pallas_call(
        functools.partial(_pass_d, n_cls=n_cls),
        grid=grid,
        in_specs=[
            pl.BlockSpec((bu, d_h), lambda i: (i, 0)),
            pl.BlockSpec((bu, 1), lambda i: (i, 0)),
        ],
        out_specs=pl.BlockSpec((bu, n_cls), lambda i: (i, 0)),
        out_shape=jax.ShapeDtypeStruct((n_u, n_cls), f32),
    )(logit, du)

    return logp, hu2
